# baseline (speedup 1.0000x reference)
import jax
import jax.numpy as jnp
from jax import lax
from jax.experimental import pallas as pl
from jax.experimental.pallas import tpu as pltpu
from jax.experimental.pallas import tpu_sc as plsc

N_NODES = 10000
D = 128
DH = D // 2
E = 160000
ROWS_PAD = 10240
DEG_W = 8
CHUNK = 128
NC = 2
NS = 16
N_CHUNKS = E // CHUNK
IDXH = 40
TLOAD = N_NODES // NS
ROWS_PER_TILE = ROWS_PAD // NS
NSLOT = 3 * NC
VECS = CHUNK // 16


def _sc_body(fu, fi, c_src, c_dst, cb_src, cb_dst, fo_src, fo_dst,
             zh, z8, o8, sums_h, degs_h,
             table, accum, degacc, zbuf, zdeg, ones_v, sidx, didx,
             rows0, rows1, gsem0, gsem1, ssem0, ssem1, dsem):
    cid = lax.axis_index("c")
    sid = lax.axis_index("s")
    pltpu.sync_copy(zh, zbuf)
    pltpu.sync_copy(z8, zdeg)
    pltpu.sync_copy(o8, ones_v)
    c0 = sid * N_CHUNKS // NS
    n_ch = (sid + 1) * N_CHUNKS // NS - c0
    pad_dst = N_NODES + sid * 8
    r0 = sid * ROWS_PER_TILE

    bufs = (rows0, rows1)
    gsems = (gsem0, gsem1)
    ssems = (ssem0, ssem1)

    def gather(r, b):
        pltpu.async_copy(table.at[sidx.at[r]], bufs[b], gsems[b])

    def gwait(r, b):
        pltpu.make_async_copy(table.at[sidx.at[r]], bufs[b], gsems[b]).wait()

    def scat(r, b):
        pltpu.async_copy(bufs[b], accum.at[didx.at[r]], ssems[b], add=True)
        pltpu.async_copy(ones_v, degacc.at[didx.at[r]], dsem, add=True)

    def swait(r, b):
        pltpu.make_async_copy(bufs[b], accum.at[didx.at[r]], ssems[b]).wait()

    jobs = ((fu, c_src, c_dst, 0, True),
            (fu, fo_src, fo_dst, 2, False),
            (fi, cb_src, cb_dst, 1, True))
    for tab, src, dst, e, load_table in jobs:
        slot = e * NC + cid
        if load_table:
            pltpu.sync_copy(
                tab.at[pl.ds(cid * N_NODES + sid * TLOAD, TLOAD)],
                table.at[pl.ds(sid * TLOAD, TLOAD)])
        for k in range(ROWS_PER_TILE // 32):
            pltpu.async_copy(zbuf, accum.at[pl.ds(r0 + k * 32, 32)], gsem0)
        for k in range(ROWS_PER_TILE // 64):
            pltpu.async_copy(zdeg, degacc.at[pl.ds(r0 + k * 64, 64)], gsem1)
        for k in range(ROWS_PER_TILE // 32):
            pltpu.make_async_copy(zbuf, accum.at[pl.ds(r0 + k * 32, 32)],
                                  gsem0).wait()
        for k in range(ROWS_PER_TILE // 64):
            pltpu.make_async_copy(zdeg, degacc.at[pl.ds(r0 + k * 64, 64)],
                                  gsem1).wait()
        plsc.subcore_barrier()

        for h, L in ((0, IDXH), (1, 39)):
            pltpu.sync_copy(src.at[pl.ds(c0 + h * IDXH, L)],
                            sidx.at[pl.ds(0, L)])
            pltpu.sync_copy(dst.at[pl.ds(c0 + h * IDXH, L)],
                            didx.at[pl.ds(0, L)])
            if h == 1:
                @pl.when(IDXH + L - 1 >= n_ch)
                def _():
                    for j in range(VECS):
                        sidx[L - 1, pl.ds(j * 16, 16)] = jnp.zeros(
                            (16,), jnp.int32)
                        didx[L - 1, pl.ds(j * 16, 16)] = jnp.full(
                            (16,), pad_dst, jnp.int32)

            gather(0, 0)

            def pair(p, carry):
                q0 = 2 * p
                q1 = q0 + 1
                gwait(q0, 0)
                scat(q0, 0)

                @pl.when(p > 0)
                def _():
                    swait(q1 - 2, 1)

                gather(q1, 1)
                gwait(q1, 1)
                scat(q1, 1)

                @pl.when(q0 + 2 < L)
                def _():
                    swait(q0, 0)
                    gather(q0 + 2, 0)

                return carry

            lax.fori_loop(0, L // 2, pair, 0)
            if L % 2:
                gwait(L - 1, 0)
                scat(L - 1, 0)
                swait(L - 2, 1)
                swait(L - 1, 0)
            else:
                swait(L - 2, 0)
                swait(L - 1, 1)

            def ddrain(r, carry):
                pltpu.make_async_copy(ones_v, degacc.at[didx.at[r]],
                                      dsem).wait()
                return carry

            lax.fori_loop(0, L, ddrain, 0)
        plsc.subcore_barrier()

        out_r0 = slot * ROWS_PAD + r0
        pltpu.sync_copy(accum.at[pl.ds(r0, ROWS_PER_TILE)],
                        sums_h.at[pl.ds(out_r0, ROWS_PER_TILE)])
        pltpu.sync_copy(degacc.at[pl.ds(r0, ROWS_PER_TILE)],
                        degs_h.at[pl.ds(out_r0, ROWS_PER_TILE)])
        plsc.subcore_barrier()


_phase1 = pl.kernel(
    _sc_body,
    out_type=(
        jax.ShapeDtypeStruct((NSLOT * ROWS_PAD, DH), jnp.float32),
        jax.ShapeDtypeStruct((NSLOT * ROWS_PAD, DEG_W), jnp.float32),
    ),
    mesh=plsc.VectorSubcoreMesh(core_axis_name="c", subcore_axis_name="s"),
    compiler_params=pltpu.CompilerParams(use_tc_tiling_on_sc=False),
    scratch_types=[
        pltpu.VMEM_SHARED((N_NODES, DH), jnp.float32),
        pltpu.VMEM_SHARED((ROWS_PAD, DH), jnp.float32),
        pltpu.VMEM_SHARED((ROWS_PAD, DEG_W), jnp.float32),
        pltpu.VMEM((32, DH), jnp.float32),
        pltpu.VMEM((64, DEG_W), jnp.float32),
        pltpu.VMEM((CHUNK, DEG_W), jnp.float32),
        pltpu.VMEM((IDXH, CHUNK), jnp.int32),
        pltpu.VMEM((IDXH, CHUNK), jnp.int32),
        pltpu.VMEM((CHUNK, DH), jnp.float32),
        pltpu.VMEM((CHUNK, DH), jnp.float32),
    ] + [pltpu.SemaphoreType.DMA] * 5,
)


def _tc_body(s_ref, d_ref, w_ref, b_ref, hu_ref, hi_ref):
    def term(e):
        d = d_ref[2 * e][:, :1]
        inv = 1.0 / jnp.maximum(d, 1.0)
        m_lo = s_ref[2 * e, 0] * inv
        m_hi = s_ref[2 * e + 1, 0] * inv
        out = jnp.dot(m_lo, w_ref[e, :DH, :],
                      preferred_element_type=jnp.float32)
        out += jnp.dot(m_hi, w_ref[e, DH:, :],
                       preferred_element_type=jnp.float32)
        return out + (d > 0.0).astype(jnp.float32) * b_ref[e][None, :]

    hi_ref[...] = term(0)
    hu_ref[...] = term(1) + term(2)


_phase2 = pl.pallas_call(
    _tc_body,
    grid=(79,),
    in_specs=[
        pl.BlockSpec((NSLOT, 1, 128, DH), lambda b: (0, b, 0, 0)),
        pl.BlockSpec((NSLOT, 128, DEG_W), lambda b: (0, b, 0)),
        pl.BlockSpec((3, D, D), lambda b: (0, 0, 0)),
        pl.BlockSpec((3, D), lambda b: (0, 0)),
    ],
    out_specs=[
        pl.BlockSpec((128, D), lambda b: (b, 0)),
        pl.BlockSpec((128, D), lambda b: (b, 0)),
    ],
    out_shape=[
        jax.ShapeDtypeStruct((N_NODES, D), jnp.float32),
        jax.ShapeDtypeStruct((N_NODES, D), jnp.float32),
    ],
)


def kernel(feat_user, feat_item, clicks_src, clicks_dst, clicked_by_src,
           clicked_by_dst, follows_src, follows_dst, W_clicks, b_clicks,
           W_clicked_by, b_clicked_by, W_follows, b_follows):
    i32 = lambda x: x.astype(jnp.int32).reshape(N_CHUNKS, CHUNK)
    zh = jnp.zeros((32, DH), jnp.float32)
    z8 = jnp.zeros((64, DEG_W), jnp.float32)
    o8 = jnp.ones((CHUNK, DEG_W), jnp.float32)
    split = lambda f: f.reshape(-1, NC, DH).transpose(1, 0, 2).reshape(-1, DH)
    sums, degs = _phase1(
        split(feat_user), split(feat_item),
        i32(clicks_src), i32(clicks_dst),
        i32(clicked_by_src), i32(clicked_by_dst),
        i32(follows_src), i32(follows_dst), zh, z8, o8)
    sums4 = sums.reshape(NSLOT, ROWS_PAD // 128, 128, DH)
    degs3 = degs.reshape(NSLOT, ROWS_PAD, DEG_W)
    wstk = jnp.stack([W_clicks, W_clicked_by, W_follows])
    bstk = jnp.stack([b_clicks, b_clicked_by, b_follows])
    return _phase2(sums4, degs3, wstk, bstk)


# speedup vs baseline: 1.2177x; 1.2177x over previous
import jax
import jax.numpy as jnp
from jax import lax
from jax.experimental import pallas as pl
from jax.experimental.pallas import tpu as pltpu
from jax.experimental.pallas import tpu_sc as plsc

N_NODES = 10000
D = 128
DH = D // 2
E = 160000
ROWS_PAD = 10240
DEG_W = 8
CHUNK = 128
NC = 2
NS = 16
N_CHUNKS = E // CHUNK
IDXH = 40
TLOAD = N_NODES // NS
ROWS_PER_TILE = ROWS_PAD // NS
NSLOT = 3 * NC
VECS = CHUNK // 16


def _sc_body(fu, fi, c_src, c_dst, cb_src, cb_dst, fo_src, fo_dst,
             zh, z8, o8, sums_h, degs_h,
             table, accum, degacc, zbuf, zdeg, ones_v, sidx, didx,
             rows0, rows1, gsem0, gsem1, ssem0, ssem1, dsem):
    cid = lax.axis_index("c")
    sid = lax.axis_index("s")
    pltpu.sync_copy(zh, zbuf)
    pltpu.sync_copy(z8, zdeg)
    pltpu.sync_copy(o8, ones_v)
    c0 = sid * N_CHUNKS // NS
    n_ch = (sid + 1) * N_CHUNKS // NS - c0
    pad_dst = N_NODES + sid * 8
    r0 = sid * ROWS_PER_TILE

    bufs = (rows0, rows1)
    gsems = (gsem0, gsem1)
    ssems = (ssem0, ssem1)

    def gather(r, b):
        pltpu.async_copy(table.at[sidx.at[r]], bufs[b], gsems[b])

    def gwait(r, b):
        pltpu.make_async_copy(table.at[sidx.at[r]], bufs[b], gsems[b]).wait()

    def scat(r, b):
        pltpu.async_copy(bufs[b], accum.at[didx.at[r]], ssems[b], add=True)
        pltpu.async_copy(ones_v, degacc.at[didx.at[r]], ssems[b], add=True)

    def swait(r, b):
        pltpu.make_async_copy(bufs[b], accum.at[didx.at[r]], ssems[b]).wait()
        pltpu.make_async_copy(ones_v, degacc.at[didx.at[r]], ssems[b]).wait()

    jobs = ((fu, c_src, c_dst, 0, True),
            (fu, fo_src, fo_dst, 2, False),
            (fi, cb_src, cb_dst, 1, True))
    for tab, src, dst, e, load_table in jobs:
        slot = e * NC + cid
        if load_table:
            pltpu.sync_copy(
                tab.at[pl.ds(cid * N_NODES + sid * TLOAD, TLOAD)],
                table.at[pl.ds(sid * TLOAD, TLOAD)])
        for k in range(ROWS_PER_TILE // 32):
            pltpu.async_copy(zbuf, accum.at[pl.ds(r0 + k * 32, 32)], gsem0)
        for k in range(ROWS_PER_TILE // 64):
            pltpu.async_copy(zdeg, degacc.at[pl.ds(r0 + k * 64, 64)], gsem1)
        for k in range(ROWS_PER_TILE // 32):
            pltpu.make_async_copy(zbuf, accum.at[pl.ds(r0 + k * 32, 32)],
                                  gsem0).wait()
        for k in range(ROWS_PER_TILE // 64):
            pltpu.make_async_copy(zdeg, degacc.at[pl.ds(r0 + k * 64, 64)],
                                  gsem1).wait()
        plsc.subcore_barrier()

        for h, L in ((0, IDXH), (1, 39)):
            pltpu.sync_copy(src.at[pl.ds(c0 + h * IDXH, L)],
                            sidx.at[pl.ds(0, L)])
            pltpu.sync_copy(dst.at[pl.ds(c0 + h * IDXH, L)],
                            didx.at[pl.ds(0, L)])
            if h == 1:
                @pl.when(IDXH + L - 1 >= n_ch)
                def _():
                    for j in range(VECS):
                        sidx[L - 1, pl.ds(j * 16, 16)] = jnp.zeros(
                            (16,), jnp.int32)
                        didx[L - 1, pl.ds(j * 16, 16)] = jnp.full(
                            (16,), pad_dst, jnp.int32)

            gather(0, 0)

            def pair(p, carry):
                q0 = 2 * p
                q1 = q0 + 1
                gwait(q0, 0)
                scat(q0, 0)

                @pl.when(p > 0)
                def _():
                    swait(q1 - 2, 1)

                gather(q1, 1)
                gwait(q1, 1)
                scat(q1, 1)

                @pl.when(q0 + 2 < L)
                def _():
                    swait(q0, 0)
                    gather(q0 + 2, 0)

                return carry

            lax.fori_loop(0, L // 2, pair, 0)
            if L % 2:
                gwait(L - 1, 0)
                scat(L - 1, 0)
                swait(L - 2, 1)
                swait(L - 1, 0)
            else:
                swait(L - 2, 0)
                swait(L - 1, 1)

        plsc.subcore_barrier()

        out_r0 = slot * ROWS_PAD + r0
        pltpu.sync_copy(accum.at[pl.ds(r0, ROWS_PER_TILE)],
                        sums_h.at[pl.ds(out_r0, ROWS_PER_TILE)])
        pltpu.sync_copy(degacc.at[pl.ds(r0, ROWS_PER_TILE)],
                        degs_h.at[pl.ds(out_r0, ROWS_PER_TILE)])
        plsc.subcore_barrier()


_phase1 = pl.kernel(
    _sc_body,
    out_type=(
        jax.ShapeDtypeStruct((NSLOT * ROWS_PAD, DH), jnp.float32),
        jax.ShapeDtypeStruct((NSLOT * ROWS_PAD, DEG_W), jnp.float32),
    ),
    mesh=plsc.VectorSubcoreMesh(core_axis_name="c", subcore_axis_name="s"),
    compiler_params=pltpu.CompilerParams(use_tc_tiling_on_sc=False),
    scratch_types=[
        pltpu.VMEM_SHARED((N_NODES, DH), jnp.float32),
        pltpu.VMEM_SHARED((ROWS_PAD, DH), jnp.float32),
        pltpu.VMEM_SHARED((ROWS_PAD, DEG_W), jnp.float32),
        pltpu.VMEM((32, DH), jnp.float32),
        pltpu.VMEM((64, DEG_W), jnp.float32),
        pltpu.VMEM((CHUNK, DEG_W), jnp.float32),
        pltpu.VMEM((IDXH, CHUNK), jnp.int32),
        pltpu.VMEM((IDXH, CHUNK), jnp.int32),
        pltpu.VMEM((CHUNK, DH), jnp.float32),
        pltpu.VMEM((CHUNK, DH), jnp.float32),
    ] + [pltpu.SemaphoreType.DMA] * 5,
)


def _tc_body(s_ref, d_ref, w_ref, b_ref, hu_ref, hi_ref):
    def term(e):
        d = d_ref[2 * e][:, :1]
        inv = 1.0 / jnp.maximum(d, 1.0)
        m_lo = s_ref[2 * e, 0] * inv
        m_hi = s_ref[2 * e + 1, 0] * inv
        out = jnp.dot(m_lo, w_ref[e, :DH, :],
                      preferred_element_type=jnp.float32)
        out += jnp.dot(m_hi, w_ref[e, DH:, :],
                       preferred_element_type=jnp.float32)
        return out + (d > 0.0).astype(jnp.float32) * b_ref[e][None, :]

    hi_ref[...] = term(0)
    hu_ref[...] = term(1) + term(2)


_phase2 = pl.pallas_call(
    _tc_body,
    grid=(79,),
    in_specs=[
        pl.BlockSpec((NSLOT, 1, 128, DH), lambda b: (0, b, 0, 0)),
        pl.BlockSpec((NSLOT, 128, DEG_W), lambda b: (0, b, 0)),
        pl.BlockSpec((3, D, D), lambda b: (0, 0, 0)),
        pl.BlockSpec((3, D), lambda b: (0, 0)),
    ],
    out_specs=[
        pl.BlockSpec((128, D), lambda b: (b, 0)),
        pl.BlockSpec((128, D), lambda b: (b, 0)),
    ],
    out_shape=[
        jax.ShapeDtypeStruct((N_NODES, D), jnp.float32),
        jax.ShapeDtypeStruct((N_NODES, D), jnp.float32),
    ],
)


def kernel(feat_user, feat_item, clicks_src, clicks_dst, clicked_by_src,
           clicked_by_dst, follows_src, follows_dst, W_clicks, b_clicks,
           W_clicked_by, b_clicked_by, W_follows, b_follows):
    i32 = lambda x: x.astype(jnp.int32).reshape(N_CHUNKS, CHUNK)
    zh = jnp.zeros((32, DH), jnp.float32)
    z8 = jnp.zeros((64, DEG_W), jnp.float32)
    o8 = jnp.ones((CHUNK, DEG_W), jnp.float32)
    split = lambda f: f.reshape(-1, NC, DH).transpose(1, 0, 2).reshape(-1, DH)
    sums, degs = _phase1(
        split(feat_user), split(feat_item),
        i32(clicks_src), i32(clicks_dst),
        i32(clicked_by_src), i32(clicked_by_dst),
        i32(follows_src), i32(follows_dst), zh, z8, o8)
    sums4 = sums.reshape(NSLOT, ROWS_PAD // 128, 128, DH)
    degs3 = degs.reshape(NSLOT, ROWS_PAD, DEG_W)
    wstk = jnp.stack([W_clicks, W_clicked_by, W_follows])
    bstk = jnp.stack([b_clicks, b_clicked_by, b_follows])
    return _phase2(sums4, degs3, wstk, bstk)


# speedup vs baseline: 1.3344x; 1.0958x over previous
import jax
import jax.numpy as jnp
from jax import lax
from jax.experimental import pallas as pl
from jax.experimental.pallas import tpu as pltpu
from jax.experimental.pallas import tpu_sc as plsc

N_NODES = 10000
D = 128
DH = D // 2
E = 160000
ROWS_PAD = 10240
DEG_W = 8
CHUNK = 128
NC = 2
NS = 16
N_CHUNKS = E // CHUNK
IDXH = 40
TLOAD = N_NODES // NS
ROWS_PER_TILE = ROWS_PAD // NS
NSLOT = 3 * NC
VECS = CHUNK // 16


def _sc_body(fu, fi, c_src, c_dst, cb_src, cb_dst, fo_src, fo_dst,
             zh, z8, o8, sums_h, degs_h,
             table, accum, degacc, zbuf, zdeg, ones_v, sidx, didx,
             rows0, rows1, gsem0, gsem1, ssem0, ssem1, dsem):
    cid = lax.axis_index("c")
    sid = lax.axis_index("s")
    pltpu.sync_copy(zh, zbuf)
    pltpu.sync_copy(z8, zdeg)
    pltpu.sync_copy(o8, ones_v)
    c0 = sid * N_CHUNKS // NS
    n_ch = (sid + 1) * N_CHUNKS // NS - c0
    pad_dst = N_NODES + sid * 8
    r0 = sid * ROWS_PER_TILE

    bufs = (rows0, rows1)
    gsems = (gsem0, gsem1)
    ssems = (ssem0, ssem1)

    def gather(r, b):
        pltpu.async_copy(table.at[sidx.at[r]], bufs[b], gsems[b])

    def gwait(r, b):
        pltpu.make_async_copy(table.at[sidx.at[r]], bufs[b], gsems[b]).wait()

    def scat(r, b):
        pltpu.async_copy(bufs[b], accum.at[didx.at[r]], ssems[b], add=True)
        pltpu.async_copy(ones_v, degacc.at[didx.at[r]], ssems[b], add=True)

    def swait(r, b):
        pltpu.make_async_copy(bufs[b], accum.at[didx.at[r]], ssems[b]).wait()
        pltpu.make_async_copy(ones_v, degacc.at[didx.at[r]], ssems[b]).wait()

    jobs = ((fu, c_src, c_dst, 0, True),
            (fu, fo_src, fo_dst, 2, False),
            (fi, cb_src, cb_dst, 1, True))
    for tab, src, dst, e, load_table in jobs:
        slot = e * NC + cid
        if load_table:
            pltpu.sync_copy(
                tab.at[pl.ds(sid * TLOAD, TLOAD), pl.ds(cid * DH, DH)],
                table.at[pl.ds(sid * TLOAD, TLOAD)])
        for k in range(ROWS_PER_TILE // 32):
            pltpu.async_copy(zbuf, accum.at[pl.ds(r0 + k * 32, 32)], gsem0)
        for k in range(ROWS_PER_TILE // 64):
            pltpu.async_copy(zdeg, degacc.at[pl.ds(r0 + k * 64, 64)], gsem1)
        for k in range(ROWS_PER_TILE // 32):
            pltpu.make_async_copy(zbuf, accum.at[pl.ds(r0 + k * 32, 32)],
                                  gsem0).wait()
        for k in range(ROWS_PER_TILE // 64):
            pltpu.make_async_copy(zdeg, degacc.at[pl.ds(r0 + k * 64, 64)],
                                  gsem1).wait()
        plsc.subcore_barrier()

        for h, L in ((0, IDXH), (1, 39)):
            pltpu.sync_copy(src.at[pl.ds(c0 + h * IDXH, L)],
                            sidx.at[pl.ds(0, L)])
            pltpu.sync_copy(dst.at[pl.ds(c0 + h * IDXH, L)],
                            didx.at[pl.ds(0, L)])
            if h == 1:
                @pl.when(IDXH + L - 1 >= n_ch)
                def _():
                    for j in range(VECS):
                        sidx[L - 1, pl.ds(j * 16, 16)] = jnp.zeros(
                            (16,), jnp.int32)
                        didx[L - 1, pl.ds(j * 16, 16)] = jnp.full(
                            (16,), pad_dst, jnp.int32)

            gather(0, 0)

            def pair(p, carry):
                q0 = 2 * p
                q1 = q0 + 1
                gwait(q0, 0)
                scat(q0, 0)

                @pl.when(p > 0)
                def _():
                    swait(q1 - 2, 1)

                gather(q1, 1)
                gwait(q1, 1)
                scat(q1, 1)

                @pl.when(q0 + 2 < L)
                def _():
                    swait(q0, 0)
                    gather(q0 + 2, 0)

                return carry

            lax.fori_loop(0, L // 2, pair, 0)
            if L % 2:
                gwait(L - 1, 0)
                scat(L - 1, 0)
                swait(L - 2, 1)
                swait(L - 1, 0)
            else:
                swait(L - 2, 0)
                swait(L - 1, 1)

        plsc.subcore_barrier()

        out_r0 = slot * ROWS_PAD + r0
        pltpu.sync_copy(accum.at[pl.ds(r0, ROWS_PER_TILE)],
                        sums_h.at[pl.ds(out_r0, ROWS_PER_TILE)])
        pltpu.sync_copy(degacc.at[pl.ds(r0, ROWS_PER_TILE)],
                        degs_h.at[pl.ds(out_r0, ROWS_PER_TILE)])
        plsc.subcore_barrier()


_phase1 = pl.kernel(
    _sc_body,
    out_type=(
        jax.ShapeDtypeStruct((NSLOT * ROWS_PAD, DH), jnp.float32),
        jax.ShapeDtypeStruct((NSLOT * ROWS_PAD, DEG_W), jnp.float32),
    ),
    mesh=plsc.VectorSubcoreMesh(core_axis_name="c", subcore_axis_name="s"),
    compiler_params=pltpu.CompilerParams(use_tc_tiling_on_sc=False),
    scratch_types=[
        pltpu.VMEM_SHARED((N_NODES, DH), jnp.float32),
        pltpu.VMEM_SHARED((ROWS_PAD, DH), jnp.float32),
        pltpu.VMEM_SHARED((ROWS_PAD, DEG_W), jnp.float32),
        pltpu.VMEM((32, DH), jnp.float32),
        pltpu.VMEM((64, DEG_W), jnp.float32),
        pltpu.VMEM((CHUNK, DEG_W), jnp.float32),
        pltpu.VMEM((IDXH, CHUNK), jnp.int32),
        pltpu.VMEM((IDXH, CHUNK), jnp.int32),
        pltpu.VMEM((CHUNK, DH), jnp.float32),
        pltpu.VMEM((CHUNK, DH), jnp.float32),
    ] + [pltpu.SemaphoreType.DMA] * 5,
)


def _tc_body(s_ref, d_ref, w_ref, b_ref, hu_ref, hi_ref):
    def term(e):
        d = d_ref[2 * e][:, :1]
        inv = 1.0 / jnp.maximum(d, 1.0)
        m_lo = s_ref[2 * e, 0] * inv
        m_hi = s_ref[2 * e + 1, 0] * inv
        out = jnp.dot(m_lo, w_ref[e, :DH, :],
                      preferred_element_type=jnp.float32)
        out += jnp.dot(m_hi, w_ref[e, DH:, :],
                       preferred_element_type=jnp.float32)
        return out + (d > 0.0).astype(jnp.float32) * b_ref[e][None, :]

    hi_ref[...] = term(0)
    hu_ref[...] = term(1) + term(2)


_phase2 = pl.pallas_call(
    _tc_body,
    grid=(79,),
    in_specs=[
        pl.BlockSpec((NSLOT, 1, 128, DH), lambda b: (0, b, 0, 0)),
        pl.BlockSpec((NSLOT, 128, DEG_W), lambda b: (0, b, 0)),
        pl.BlockSpec((3, D, D), lambda b: (0, 0, 0)),
        pl.BlockSpec((3, D), lambda b: (0, 0)),
    ],
    out_specs=[
        pl.BlockSpec((128, D), lambda b: (b, 0)),
        pl.BlockSpec((128, D), lambda b: (b, 0)),
    ],
    out_shape=[
        jax.ShapeDtypeStruct((N_NODES, D), jnp.float32),
        jax.ShapeDtypeStruct((N_NODES, D), jnp.float32),
    ],
)


def kernel(feat_user, feat_item, clicks_src, clicks_dst, clicked_by_src,
           clicked_by_dst, follows_src, follows_dst, W_clicks, b_clicks,
           W_clicked_by, b_clicked_by, W_follows, b_follows):
    i32 = lambda x: x.astype(jnp.int32).reshape(N_CHUNKS, CHUNK)
    zh = jnp.zeros((32, DH), jnp.float32)
    z8 = jnp.zeros((64, DEG_W), jnp.float32)
    o8 = jnp.ones((CHUNK, DEG_W), jnp.float32)
    sums, degs = _phase1(
        feat_user, feat_item,
        i32(clicks_src), i32(clicks_dst),
        i32(clicked_by_src), i32(clicked_by_dst),
        i32(follows_src), i32(follows_dst), zh, z8, o8)
    sums4 = sums.reshape(NSLOT, ROWS_PAD // 128, 128, DH)
    degs3 = degs.reshape(NSLOT, ROWS_PAD, DEG_W)
    wstk = jnp.stack([W_clicks, W_clicked_by, W_follows])
    bstk = jnp.stack([b_clicks, b_clicked_by, b_follows])
    return _phase2(sums4, degs3, wstk, bstk)


# speedup vs baseline: 1.3645x; 1.0225x over previous
import jax
import jax.numpy as jnp
from jax import lax
from jax.experimental import pallas as pl
from jax.experimental.pallas import tpu as pltpu
from jax.experimental.pallas import tpu_sc as plsc

N_NODES = 10000
D = 128
DH = D // 2
E = 160000
ROWS_PAD = 10240
DEG_W = 8
CHUNK = 128
NC = 2
NS = 16
N_CHUNKS = E // CHUNK
IDXH = 40
TLOAD = N_NODES // NS
ROWS_PER_TILE = ROWS_PAD // NS
NSLOT = 3 * NC
VECS = CHUNK // 16


def _sc_body(fu, fi, c_src, c_dst, cb_src, cb_dst, fo_src, fo_dst,
             zh, z8, o8, sums_h, degs_h,
             table, accum, degacc, zbuf, zdeg, ones_v, sidx, didx,
             rows0, rows1, gsem0, gsem1, ssem0, ssem1, dsem):
    cid = lax.axis_index("c")
    sid = lax.axis_index("s")
    pltpu.sync_copy(zh, zbuf)
    pltpu.sync_copy(z8, zdeg)
    pltpu.sync_copy(o8, ones_v)
    c0 = sid * N_CHUNKS // NS
    n_ch = (sid + 1) * N_CHUNKS // NS - c0
    pad_dst = N_NODES + sid * 8
    r0 = sid * ROWS_PER_TILE

    bufs = (rows0, rows1)
    gsems = (gsem0, gsem1)
    ssems = (ssem0, ssem1)

    def gather(r, b):
        pltpu.async_copy(table.at[sidx.at[r]], bufs[b], gsems[b])

    def gwait(r, b):
        pltpu.make_async_copy(table.at[sidx.at[r]], bufs[b], gsems[b]).wait()

    def scat(r, b):
        pltpu.async_copy(bufs[b], accum.at[didx.at[r]], ssems[b], add=True)

        @pl.when(cid == b)
        def _():
            pltpu.async_copy(ones_v, degacc.at[didx.at[r]], ssems[b],
                             add=True)

    def swait(r, b):
        pltpu.make_async_copy(bufs[b], accum.at[didx.at[r]], ssems[b]).wait()

        @pl.when(cid == b)
        def _():
            pltpu.make_async_copy(ones_v, degacc.at[didx.at[r]],
                                  ssems[b]).wait()

    jobs = ((fu, c_src, c_dst, 0, True),
            (fu, fo_src, fo_dst, 2, False),
            (fi, cb_src, cb_dst, 1, True))
    for tab, src, dst, e, load_table in jobs:
        slot = e * NC + cid
        if load_table:
            pltpu.sync_copy(
                tab.at[pl.ds(sid * TLOAD, TLOAD), pl.ds(cid * DH, DH)],
                table.at[pl.ds(sid * TLOAD, TLOAD)])
        for k in range(ROWS_PER_TILE // 32):
            pltpu.async_copy(zbuf, accum.at[pl.ds(r0 + k * 32, 32)], gsem0)
        for k in range(ROWS_PER_TILE // 64):
            pltpu.async_copy(zdeg, degacc.at[pl.ds(r0 + k * 64, 64)], gsem1)
        for k in range(ROWS_PER_TILE // 32):
            pltpu.make_async_copy(zbuf, accum.at[pl.ds(r0 + k * 32, 32)],
                                  gsem0).wait()
        for k in range(ROWS_PER_TILE // 64):
            pltpu.make_async_copy(zdeg, degacc.at[pl.ds(r0 + k * 64, 64)],
                                  gsem1).wait()
        plsc.subcore_barrier()

        for h, L in ((0, IDXH), (1, 39)):
            pltpu.sync_copy(src.at[pl.ds(c0 + h * IDXH, L)],
                            sidx.at[pl.ds(0, L)])
            pltpu.sync_copy(dst.at[pl.ds(c0 + h * IDXH, L)],
                            didx.at[pl.ds(0, L)])
            if h == 1:
                @pl.when(IDXH + L - 1 >= n_ch)
                def _():
                    for j in range(VECS):
                        sidx[L - 1, pl.ds(j * 16, 16)] = jnp.zeros(
                            (16,), jnp.int32)
                        didx[L - 1, pl.ds(j * 16, 16)] = jnp.full(
                            (16,), pad_dst, jnp.int32)

            gather(0, 0)

            def pair(p, carry):
                q0 = 2 * p
                q1 = q0 + 1
                gwait(q0, 0)
                scat(q0, 0)

                @pl.when(p > 0)
                def _():
                    swait(q1 - 2, 1)

                gather(q1, 1)
                gwait(q1, 1)
                scat(q1, 1)

                @pl.when(q0 + 2 < L)
                def _():
                    swait(q0, 0)
                    gather(q0 + 2, 0)

                return carry

            lax.fori_loop(0, L // 2, pair, 0)
            if L % 2:
                gwait(L - 1, 0)
                scat(L - 1, 0)
                swait(L - 2, 1)
                swait(L - 1, 0)
            else:
                swait(L - 2, 0)
                swait(L - 1, 1)

        plsc.subcore_barrier()

        out_r0 = slot * ROWS_PAD + r0
        pltpu.sync_copy(accum.at[pl.ds(r0, ROWS_PER_TILE)],
                        sums_h.at[pl.ds(out_r0, ROWS_PER_TILE)])
        pltpu.sync_copy(degacc.at[pl.ds(r0, ROWS_PER_TILE)],
                        degs_h.at[pl.ds(out_r0, ROWS_PER_TILE)])
        plsc.subcore_barrier()


_phase1 = pl.kernel(
    _sc_body,
    out_type=(
        jax.ShapeDtypeStruct((NSLOT * ROWS_PAD, DH), jnp.float32),
        jax.ShapeDtypeStruct((NSLOT * ROWS_PAD, DEG_W), jnp.float32),
    ),
    mesh=plsc.VectorSubcoreMesh(core_axis_name="c", subcore_axis_name="s"),
    compiler_params=pltpu.CompilerParams(use_tc_tiling_on_sc=False),
    scratch_types=[
        pltpu.VMEM_SHARED((N_NODES, DH), jnp.float32),
        pltpu.VMEM_SHARED((ROWS_PAD, DH), jnp.float32),
        pltpu.VMEM_SHARED((ROWS_PAD, DEG_W), jnp.float32),
        pltpu.VMEM((32, DH), jnp.float32),
        pltpu.VMEM((64, DEG_W), jnp.float32),
        pltpu.VMEM((CHUNK, DEG_W), jnp.float32),
        pltpu.VMEM((IDXH, CHUNK), jnp.int32),
        pltpu.VMEM((IDXH, CHUNK), jnp.int32),
        pltpu.VMEM((CHUNK, DH), jnp.float32),
        pltpu.VMEM((CHUNK, DH), jnp.float32),
    ] + [pltpu.SemaphoreType.DMA] * 5,
)


def _tc_body(s_ref, d_ref, w_ref, b_ref, hu_ref, hi_ref):
    def term(e):
        d = d_ref[2 * e][:, :1] + d_ref[2 * e + 1][:, :1]
        inv = 1.0 / jnp.maximum(d, 1.0)
        m_lo = s_ref[2 * e, 0] * inv
        m_hi = s_ref[2 * e + 1, 0] * inv
        out = jnp.dot(m_lo, w_ref[e, :DH, :],
                      preferred_element_type=jnp.float32)
        out += jnp.dot(m_hi, w_ref[e, DH:, :],
                       preferred_element_type=jnp.float32)
        return out + (d > 0.0).astype(jnp.float32) * b_ref[e][None, :]

    hi_ref[...] = term(0)
    hu_ref[...] = term(1) + term(2)


_phase2 = pl.pallas_call(
    _tc_body,
    grid=(79,),
    in_specs=[
        pl.BlockSpec((NSLOT, 1, 128, DH), lambda b: (0, b, 0, 0)),
        pl.BlockSpec((NSLOT, 128, DEG_W), lambda b: (0, b, 0)),
        pl.BlockSpec((3, D, D), lambda b: (0, 0, 0)),
        pl.BlockSpec((3, D), lambda b: (0, 0)),
    ],
    out_specs=[
        pl.BlockSpec((128, D), lambda b: (b, 0)),
        pl.BlockSpec((128, D), lambda b: (b, 0)),
    ],
    out_shape=[
        jax.ShapeDtypeStruct((N_NODES, D), jnp.float32),
        jax.ShapeDtypeStruct((N_NODES, D), jnp.float32),
    ],
)


def kernel(feat_user, feat_item, clicks_src, clicks_dst, clicked_by_src,
           clicked_by_dst, follows_src, follows_dst, W_clicks, b_clicks,
           W_clicked_by, b_clicked_by, W_follows, b_follows):
    i32 = lambda x: x.astype(jnp.int32).reshape(N_CHUNKS, CHUNK)
    zh = jnp.zeros((32, DH), jnp.float32)
    z8 = jnp.zeros((64, DEG_W), jnp.float32)
    o8 = jnp.ones((CHUNK, DEG_W), jnp.float32)
    sums, degs = _phase1(
        feat_user, feat_item,
        i32(clicks_src), i32(clicks_dst),
        i32(clicked_by_src), i32(clicked_by_dst),
        i32(follows_src), i32(follows_dst), zh, z8, o8)
    sums4 = sums.reshape(NSLOT, ROWS_PAD // 128, 128, DH)
    degs3 = degs.reshape(NSLOT, ROWS_PAD, DEG_W)
    wstk = jnp.stack([W_clicks, W_clicked_by, W_follows])
    bstk = jnp.stack([b_clicks, b_clicked_by, b_follows])
    return _phase2(sums4, degs3, wstk, bstk)
